# baseline (speedup 1.0000x reference)
import functools

import jax
import jax.numpy as jnp
from jax import lax
from jax.experimental import pallas as pl
from jax.experimental.pallas import tpu as pltpu
from jax.experimental.pallas import tpu_sc as plsc


def kernel(inputs, table):
    B, S, D = inputs.shape
    info = plsc.get_sparse_core_info()
    NC, NS, L = info.num_cores, info.num_subcores, info.num_lanes
    NW = NC * NS
    SPW = S // NW
    CS = 32
    NCHUNK = SPW // CS
    CW = CS * D
    NV = CW // L

    x2 = inputs.reshape(B, S * D)
    t1 = table.reshape(S * D)

    mesh = plsc.VectorSubcoreMesh(core_axis_name="c", subcore_axis_name="s")

    @functools.partial(
        pl.kernel,
        mesh=mesh,
        out_type=jax.ShapeDtypeStruct((B, S * D), jnp.float32),
        scratch_types=[
            pltpu.VMEM((CW,), jnp.float32),
            pltpu.VMEM((CW,), jnp.float32),
        ],
    )
    def k(x_hbm, t_hbm, o_hbm, tbuf, xbuf):
        wid = lax.axis_index("s") * NC + lax.axis_index("c")
        base = wid * SPW * D

        def chunk_body(c, carry):
            off = base + c * CW
            pltpu.sync_copy(t_hbm.at[pl.ds(off, CW)], tbuf)
            for b in range(B):
                pltpu.sync_copy(x_hbm.at[b, pl.ds(off, CW)], xbuf)

                def vbody(i, carry2):
                    sl = pl.ds(i * L, L)
                    xbuf[sl] = xbuf[sl] + tbuf[sl]
                    return carry2

                lax.fori_loop(0, NV, vbody, 0, unroll=8)
                pltpu.sync_copy(xbuf, o_hbm.at[b, pl.ds(off, CW)])
            return carry

        lax.fori_loop(0, NCHUNK, chunk_body, 0)

    out = k(x2, t1)
    return out.reshape(B, S, D)


# speedup vs baseline: 2.8264x; 2.8264x over previous
import functools

import jax
import jax.numpy as jnp
from jax import lax
from jax.experimental import pallas as pl
from jax.experimental.pallas import tpu as pltpu
from jax.experimental.pallas import tpu_sc as plsc


def kernel(inputs, table):
    B, S, D = inputs.shape
    info = plsc.get_sparse_core_info()
    NC, NS, L = info.num_cores, info.num_subcores, info.num_lanes
    NW = NC * NS
    SPW = S // NW
    CS = 32
    NCHUNK = SPW // CS
    NVC = D // L

    mesh = plsc.VectorSubcoreMesh(core_axis_name="c", subcore_axis_name="s")

    @functools.partial(
        pl.kernel,
        mesh=mesh,
        out_type=jax.ShapeDtypeStruct((B, S, D), jnp.float32),
        scratch_types=[
            pltpu.VMEM((CS, D), jnp.float32),
            pltpu.VMEM((CS, D), jnp.float32),
        ],
    )
    def k(x_hbm, t_hbm, o_hbm, tbuf, xbuf):
        wid = lax.axis_index("s") * NC + lax.axis_index("c")
        base = wid * SPW

        def chunk_body(c, carry):
            s0 = base + c * CS
            pltpu.sync_copy(t_hbm.at[pl.ds(s0, CS)], tbuf)
            for b in range(B):
                pltpu.sync_copy(x_hbm.at[b, pl.ds(s0, CS)], xbuf)

                def row_body(r, carry2):
                    for cc in range(NVC):
                        sl = pl.ds(cc * L, L)
                        xbuf[r, sl] = xbuf[r, sl] + tbuf[r, sl]
                    return carry2

                lax.fori_loop(0, CS, row_body, 0)
                pltpu.sync_copy(xbuf, o_hbm.at[b, pl.ds(s0, CS)])
            return carry

        lax.fori_loop(0, NCHUNK, chunk_body, 0)

    return k(inputs, table)


# speedup vs baseline: 7.2032x; 2.5485x over previous
import jax
import jax.numpy as jnp
from jax.experimental import pallas as pl
from jax.experimental.pallas import tpu as pltpu


def kernel(inputs, table):
    B, S, D = inputs.shape
    BS = 2048

    def body(x_ref, t_ref, o_ref):
        o_ref[...] = x_ref[...] + t_ref[...]

    return pl.pallas_call(
        body,
        grid=(S // BS, B),
        in_specs=[
            pl.BlockSpec((1, BS, D), lambda s, b: (b, s, 0)),
            pl.BlockSpec((BS, D), lambda s, b: (s, 0)),
        ],
        out_specs=pl.BlockSpec((1, BS, D), lambda s, b: (b, s, 0)),
        out_shape=jax.ShapeDtypeStruct((B, S, D), inputs.dtype),
        compiler_params=pltpu.CompilerParams(
            dimension_semantics=("arbitrary", "arbitrary"),
        ),
    )(inputs, table)
